# Initial kernel scaffold; baseline (speedup 1.0000x reference)
import jax
import jax.numpy as jnp
from jax.experimental import pallas as pl


def kernel(nfeat, edge_index, efeat, W_node, b_node, W_edge, b_edge, res_weight):
    raise NotImplementedError("write your pallas kernel here")


# speedup vs baseline: 11.3113x; 11.3113x over previous
import functools

import jax
import jax.numpy as jnp
from jax import lax
from jax.experimental import pallas as pl
from jax.experimental.pallas import tpu as pltpu
from jax.experimental.pallas import tpu_sc as plsc

N = 10000
E = 320000
D = 128
DE = 16
DW = D + 16

NC = 2
NS = 16
NW = NC * NS
EPW = E // NW
B = 80
NB = EPW // B
RPT = 632
RLAST = N - (NS - 1) * RPT

_mesh = plsc.VectorSubcoreMesh(core_axis_name="c", subcore_axis_name="s")


@functools.partial(
    pl.kernel,
    mesh=_mesh,
    out_type=jax.ShapeDtypeStruct((NW, N), jnp.float32),
    scratch_types=[
        pltpu.VMEM((EPW,), jnp.int32),
        pltpu.VMEM((N,), jnp.float32),
    ],
    compiler_params=pltpu.CompilerParams(needs_layout_passes=False),
)
def _deg_kernel(dst_hbm, out_hbm, dst_v, hist_v):
    cid = lax.axis_index("c")
    sid = lax.axis_index("s")
    wid = cid * NS + sid
    pltpu.sync_copy(dst_hbm.at[pl.ds(wid * EPW, EPW)], dst_v)

    zeros = jnp.zeros((16,), jnp.float32)

    def zbody(i, carry):
        hist_v[pl.ds(i * 16, 16)] = zeros
        return carry

    lax.fori_loop(0, N // 16, zbody, 0)

    ones = jnp.ones((16,), jnp.float32)

    def body(g, carry):
        idx = dst_v[pl.ds(g * 16, 16)]
        plsc.addupdate_scatter(hist_v, [idx], ones)
        return carry

    lax.fori_loop(0, EPW // 16, body, 0)
    pltpu.sync_copy(hist_v, out_hbm.at[wid])


_BLK = 1000
_GRID = N // _BLK


def _prep_body(degp, nfeat, wn, bn, rw, y, resid, norm):
    degs = jnp.sum(degp[...], axis=0) + 1.0
    nv = lax.rsqrt(degs)
    nv = nv * (1.5 - 0.5 * degs * nv * nv)
    inv = nv * nv
    x = nfeat[...]
    h = jnp.dot(x, wn[...], preferred_element_type=jnp.float32) + bn[...]
    y[:, :D] = nv[:, None] * x
    y[:, D:] = jnp.concatenate(
        [nv[:, None], jnp.zeros((N, DW - D - 1), jnp.float32)], axis=1)
    resid[...] = (h + rw[...]) * inv[:, None]
    norm[...] = nv[:, None]


def _prep(degp, nfeat, wn, bn, rw):
    return pl.pallas_call(
        _prep_body,
        out_shape=[
            jax.ShapeDtypeStruct((N, DW), jnp.float32),
            jax.ShapeDtypeStruct((N, D), jnp.float32),
            jax.ShapeDtypeStruct((N, 1), jnp.float32),
        ],
    )(degp, nfeat, wn, bn, rw)


@functools.partial(
    pl.kernel,
    mesh=_mesh,
    out_type=(
        jax.ShapeDtypeStruct((NC * N, DW), jnp.float32),
        jax.ShapeDtypeStruct((NC * N, DE), jnp.float32),
    ),
    scratch_types=[
        pltpu.VMEM((2, B), jnp.int32),
        pltpu.VMEM((2, B), jnp.int32),
        pltpu.VMEM((2, B, DW), jnp.float32),
        pltpu.VMEM((B, DE), jnp.float32),
        pltpu.VMEM((B, DE), jnp.float32),
        pltpu.VMEM_SHARED((N, DW), jnp.float32),
        pltpu.VMEM_SHARED((N, DE), jnp.float32),
        pltpu.SemaphoreType.DMA,
    ],
    compiler_params=pltpu.CompilerParams(needs_layout_passes=False, use_tc_tiling_on_sc=False),
)
def _edge_kernel(src_hbm, dst_hbm, y_hbm, efeat_hbm, z144_hbm, z16_hbm,
                 g_out, ef_out, src_v, dst_v, ybuf, efbuf, build,
                 g_acc, ef_acc, sem_g):
    cid = lax.axis_index("c")
    sid = lax.axis_index("s")
    wid = cid * NS + sid
    ebase = wid * EPW

    rbase = pl.multiple_of(sid * RPT, 8)

    @pl.when(sid < NS - 1)
    def _():
        pltpu.sync_copy(z144_hbm, g_acc.at[pl.ds(rbase, RPT)])
        pltpu.sync_copy(z16_hbm, ef_acc.at[pl.ds(rbase, RPT)])

    @pl.when(sid == NS - 1)
    def _():
        pltpu.sync_copy(z144_hbm.at[pl.ds(0, RLAST)],
                        g_acc.at[pl.ds((NS - 1) * RPT, RLAST)])
        pltpu.sync_copy(z16_hbm.at[pl.ds(0, RLAST)],
                        ef_acc.at[pl.ds((NS - 1) * RPT, RLAST)])

    plsc.subcore_barrier()

    pltpu.sync_copy(src_hbm.at[pl.ds(ebase, B)], src_v.at[0])
    pltpu.sync_copy(dst_hbm.at[pl.ds(ebase, B)], dst_v.at[0])
    pltpu.async_copy(y_hbm.at[src_v.at[0]], ybuf.at[0], sem_g)

    iota16 = lax.iota(jnp.int32, 16)
    col_n = jnp.full((16,), D, jnp.int32)

    def body(b, carry):
        s = lax.rem(b, 2)

        @pl.when(b + 1 < NB)
        def _():
            nbase = ebase + (b + 1) * B
            pltpu.sync_copy(src_hbm.at[pl.ds(nbase, B)], src_v.at[1 - s])
            pltpu.sync_copy(dst_hbm.at[pl.ds(nbase, B)], dst_v.at[1 - s])

        pltpu.make_async_copy(y_hbm.at[src_v.at[s]], ybuf.at[s], sem_g).wait()

        @pl.when(b + 1 < NB)
        def _():
            pltpu.async_copy(y_hbm.at[src_v.at[1 - s]], ybuf.at[1 - s], sem_g)

        pltpu.sync_copy(efeat_hbm.at[pl.ds(ebase + b * B, B)], efbuf)

        for g in range(B // 16):
            rows = iota16 + g * 16
            sv = plsc.load_gather(ybuf.at[s], [rows, col_n])
            for j in range(DE):
                jv = jnp.full((16,), j, jnp.int32)
                col = plsc.load_gather(efbuf, [rows, jv])
                plsc.store_scatter(build, [rows, jv], col * sv)

        pltpu.sync_copy(ybuf.at[s], g_acc.at[dst_v.at[s]], add=True)
        pltpu.sync_copy(build, ef_acc.at[dst_v.at[s]], add=True)
        return carry

    lax.fori_loop(0, NB, body, 0)

    plsc.subcore_barrier()
    obase = pl.multiple_of(cid * N + sid * RPT, 8)

    @pl.when(sid < NS - 1)
    def _():
        pltpu.sync_copy(g_acc.at[pl.ds(rbase, RPT)],
                        g_out.at[pl.ds(obase, RPT)])
        pltpu.sync_copy(ef_acc.at[pl.ds(rbase, RPT)],
                        ef_out.at[pl.ds(obase, RPT)])

    @pl.when(sid == NS - 1)
    def _():
        lb = (NS - 1) * RPT
        pltpu.sync_copy(g_acc.at[pl.ds(lb, RLAST)],
                        g_out.at[pl.ds(cid * N + lb, RLAST)])
        pltpu.sync_copy(ef_acc.at[pl.ds(lb, RLAST)],
                        ef_out.at[pl.ds(cid * N + lb, RLAST)])


def _final_body(g0, g1, ef0, ef1, norm, resid, wn, we, bn, be, out):
    g = g0[...] + g1[...]
    ef = ef0[...] + ef1[...]
    t = g[:, D:D + 1]
    acc = jnp.dot(g[:, :D], wn[...], preferred_element_type=jnp.float32)
    acc += jnp.dot(ef, we[...], preferred_element_type=jnp.float32)
    acc += t * (bn[...] + be[...])
    out[...] = norm[...] * acc + resid[...]


def _final(gp, efp, norm, resid, wn, we, bn, be):
    return pl.pallas_call(
        _final_body,
        out_shape=jax.ShapeDtypeStruct((N, D), jnp.float32),
    )(gp[:N], gp[N:], efp[:N], efp[N:], norm, resid, wn, we, bn, be)


def kernel(nfeat, edge_index, efeat, W_node, b_node, W_edge, b_edge,
           res_weight):
    src = edge_index[0]
    dst = edge_index[1]

    degp = _deg_kernel(dst)
    y, resid, norm = _prep(degp, nfeat, W_node, b_node.reshape(1, D),
                           res_weight)
    z144 = jnp.zeros((RPT, DW), jnp.float32)
    z16 = jnp.zeros((RPT, DE), jnp.float32)
    gp, efp = _edge_kernel(src, dst, y, efeat, z144, z16)
    return _final(gp, efp, norm, resid, W_node, W_edge,
                  b_node.reshape(1, D), b_edge.reshape(1, D))


# speedup vs baseline: 15.5637x; 1.3759x over previous
import functools

import jax
import jax.numpy as jnp
from jax import lax
from jax.experimental import pallas as pl
from jax.experimental.pallas import tpu as pltpu
from jax.experimental.pallas import tpu_sc as plsc

N = 10000
E = 320000
D = 128
DE = 16
DW = D + 16

NC = 2
NS = 16
NW = NC * NS
EPW = E // NW
B = 80
NB = EPW // B
CH = 5
RPT = 632
RLAST = N - (NS - 1) * RPT

_mesh = plsc.VectorSubcoreMesh(core_axis_name="c", subcore_axis_name="s")


@functools.partial(
    pl.kernel,
    mesh=_mesh,
    out_type=jax.ShapeDtypeStruct((NW, N), jnp.float32),
    scratch_types=[
        pltpu.VMEM((EPW,), jnp.int32),
        pltpu.VMEM((N,), jnp.float32),
    ],
    compiler_params=pltpu.CompilerParams(needs_layout_passes=False),
)
def _deg_kernel(dst_hbm, out_hbm, dst_v, hist_v):
    cid = lax.axis_index("c")
    sid = lax.axis_index("s")
    wid = cid * NS + sid
    pltpu.sync_copy(dst_hbm.at[pl.ds(wid * EPW, EPW)], dst_v)

    zeros = jnp.zeros((16,), jnp.float32)

    def zbody(i, carry):
        hist_v[pl.ds(i * 16, 16)] = zeros
        return carry

    lax.fori_loop(0, N // 16, zbody, 0)

    ones = jnp.ones((16,), jnp.float32)

    def body(g, carry):
        idx = dst_v[pl.ds(g * 16, 16)]
        plsc.addupdate_scatter(hist_v, [idx], ones)
        return carry

    lax.fori_loop(0, EPW // 16, body, 0)
    pltpu.sync_copy(hist_v, out_hbm.at[wid])


_BLK = 1000
_GRID = N // _BLK


def _prep_body(degp, nfeat, wn, bn, rw, y, resid, norm):
    degs = jnp.sum(degp[...], axis=0) + 1.0
    nv = lax.rsqrt(degs)
    nv = nv * (1.5 - 0.5 * degs * nv * nv)
    inv = nv * nv
    x = nfeat[...]
    h = jnp.dot(x, wn[...], preferred_element_type=jnp.float32) + bn[...]
    y[:, :D] = nv[:, None] * x
    y[:, D:] = jnp.concatenate(
        [nv[:, None], jnp.zeros((N, DW - D - 1), jnp.float32)], axis=1)
    resid[...] = (h + rw[...]) * inv[:, None]
    norm[...] = nv[:, None]


def _prep(degp, nfeat, wn, bn, rw):
    return pl.pallas_call(
        _prep_body,
        out_shape=[
            jax.ShapeDtypeStruct((N, DW), jnp.float32),
            jax.ShapeDtypeStruct((N, D), jnp.float32),
            jax.ShapeDtypeStruct((N, 1), jnp.float32),
        ],
    )(degp, nfeat, wn, bn, rw)


@functools.partial(
    pl.kernel,
    mesh=_mesh,
    out_type=(
        jax.ShapeDtypeStruct((NC * N, DW), jnp.float32),
        jax.ShapeDtypeStruct((NC * N, DE), jnp.float32),
    ),
    scratch_types=[
        pltpu.VMEM((2, CH, B), jnp.int32),
        pltpu.VMEM((2, CH, B), jnp.int32),
        pltpu.VMEM((2, B, DW), jnp.float32),
        pltpu.VMEM((2, B, DE), jnp.float32),
        pltpu.VMEM((2, B, DE), jnp.float32),
        pltpu.VMEM_SHARED((N, DW), jnp.float32),
        pltpu.VMEM_SHARED((N, DE), jnp.float32),
        pltpu.SemaphoreType.DMA,
        pltpu.SemaphoreType.DMA,
        pltpu.SemaphoreType.DMA,
    ],
    compiler_params=pltpu.CompilerParams(needs_layout_passes=False, use_tc_tiling_on_sc=False),
)
def _edge_kernel(src_hbm, dst_hbm, y_hbm, efeat_hbm, z144_hbm, z16_hbm,
                 g_out, ef_out, src_v, dst_v, ybuf, efbuf, build,
                 g_acc, ef_acc, sem_g, sem_e, sem_a):
    cid = lax.axis_index("c")
    sid = lax.axis_index("s")
    wid = cid * NS + sid
    ebase = wid * EPW

    rbase = pl.multiple_of(sid * RPT, 8)

    @pl.when(sid < NS - 1)
    def _():
        pltpu.sync_copy(z144_hbm, g_acc.at[pl.ds(rbase, RPT)])
        pltpu.sync_copy(z16_hbm, ef_acc.at[pl.ds(rbase, RPT)])

    @pl.when(sid == NS - 1)
    def _():
        pltpu.sync_copy(z144_hbm.at[pl.ds(0, RLAST)],
                        g_acc.at[pl.ds((NS - 1) * RPT, RLAST)])
        pltpu.sync_copy(z16_hbm.at[pl.ds(0, RLAST)],
                        ef_acc.at[pl.ds((NS - 1) * RPT, RLAST)])

    plsc.subcore_barrier()

    pltpu.sync_copy(src_hbm.at[wid, pl.ds(0, CH)], src_v.at[0])
    pltpu.sync_copy(dst_hbm.at[wid, pl.ds(0, CH)], dst_v.at[0])
    pltpu.async_copy(y_hbm.at[src_v.at[0, 0]], ybuf.at[0], sem_g)
    pltpu.async_copy(efeat_hbm.at[pl.ds(ebase, B)], efbuf.at[0], sem_e)

    iota16 = lax.iota(jnp.int32, 16)
    col_n = jnp.full((16,), D, jnp.int32)

    def body(b, carry):
        s = lax.rem(b, 2)
        c = lax.div(b, CH)
        cs = lax.rem(c, 2)
        bi = lax.rem(b, CH)

        pltpu.make_async_copy(y_hbm.at[src_v.at[cs, bi]], ybuf.at[s],
                              sem_g).wait()
        pltpu.make_async_copy(efeat_hbm.at[pl.ds(ebase + b * B, B)],
                              efbuf.at[s], sem_e).wait()

        @pl.when(b > 0)
        def _():
            pltpu.make_async_copy(ybuf.at[1 - s], g_acc.at[dst_v.at[cs, bi]],
                                  sem_a).wait()
            pltpu.make_async_copy(build.at[1 - s],
                                  ef_acc.at[dst_v.at[cs, bi]], sem_a).wait()

        @pl.when((bi == CH - 1) & (b + 1 < NB))
        def _():
            pltpu.sync_copy(src_hbm.at[wid, pl.ds((c + 1) * CH, CH)],
                            src_v.at[1 - cs])
            pltpu.sync_copy(dst_hbm.at[wid, pl.ds((c + 1) * CH, CH)],
                            dst_v.at[1 - cs])

        @pl.when(b + 1 < NB)
        def _():
            c1 = lax.div(b + 1, CH)
            pltpu.async_copy(
                y_hbm.at[src_v.at[lax.rem(c1, 2), lax.rem(b + 1, CH)]],
                ybuf.at[1 - s], sem_g)
            pltpu.async_copy(efeat_hbm.at[pl.ds(ebase + (b + 1) * B, B)],
                             efbuf.at[1 - s], sem_e)

        for g in range(B // 16):
            rows = iota16 + g * 16
            sv = plsc.load_gather(ybuf.at[s], [rows, col_n])
            for j in range(DE):
                jv = jnp.full((16,), j, jnp.int32)
                col = plsc.load_gather(efbuf.at[s], [rows, jv])
                plsc.store_scatter(build.at[s], [rows, jv], col * sv)

        pltpu.async_copy(ybuf.at[s], g_acc.at[dst_v.at[cs, bi]], sem_a,
                         add=True)
        pltpu.async_copy(build.at[s], ef_acc.at[dst_v.at[cs, bi]], sem_a,
                         add=True)
        return carry

    lax.fori_loop(0, NB, body, 0)

    ls = lax.rem(NB - 1, 2)
    lcs = lax.rem(lax.div(NB - 1, CH), 2)
    lbi = lax.rem(NB - 1, CH)
    pltpu.make_async_copy(ybuf.at[ls], g_acc.at[dst_v.at[lcs, lbi]],
                          sem_a).wait()
    pltpu.make_async_copy(build.at[ls], ef_acc.at[dst_v.at[lcs, lbi]],
                          sem_a).wait()

    plsc.subcore_barrier()
    obase = pl.multiple_of(cid * N + sid * RPT, 8)

    @pl.when(sid < NS - 1)
    def _():
        pltpu.sync_copy(g_acc.at[pl.ds(rbase, RPT)],
                        g_out.at[pl.ds(obase, RPT)])
        pltpu.sync_copy(ef_acc.at[pl.ds(rbase, RPT)],
                        ef_out.at[pl.ds(obase, RPT)])

    @pl.when(sid == NS - 1)
    def _():
        lb = (NS - 1) * RPT
        pltpu.sync_copy(g_acc.at[pl.ds(lb, RLAST)],
                        g_out.at[pl.ds(cid * N + lb, RLAST)])
        pltpu.sync_copy(ef_acc.at[pl.ds(lb, RLAST)],
                        ef_out.at[pl.ds(cid * N + lb, RLAST)])


def _final_body(g0, g1, ef0, ef1, norm, resid, wn, we, bn, be, out):
    g = g0[...] + g1[...]
    ef = ef0[...] + ef1[...]
    t = g[:, D:D + 1]
    acc = jnp.dot(g[:, :D], wn[...], preferred_element_type=jnp.float32)
    acc += jnp.dot(ef, we[...], preferred_element_type=jnp.float32)
    acc += t * (bn[...] + be[...])
    out[...] = norm[...] * acc + resid[...]


def _final(gp, efp, norm, resid, wn, we, bn, be):
    return pl.pallas_call(
        _final_body,
        out_shape=jax.ShapeDtypeStruct((N, D), jnp.float32),
    )(gp[:N], gp[N:], efp[:N], efp[N:], norm, resid, wn, we, bn, be)


def kernel(nfeat, edge_index, efeat, W_node, b_node, W_edge, b_edge,
           res_weight):
    src = edge_index[0]
    dst = edge_index[1]

    degp = _deg_kernel(dst)
    y, resid, norm = _prep(degp, nfeat, W_node, b_node.reshape(1, D),
                           res_weight)
    z144 = jnp.zeros((RPT, DW), jnp.float32)
    z16 = jnp.zeros((RPT, DE), jnp.float32)
    gp, efp = _edge_kernel(src.reshape(NW, NB, B), dst.reshape(NW, NB, B),
                           y, efeat, z144, z16)
    return _final(gp, efp, norm, resid, W_node, W_edge,
                  b_node.reshape(1, D), b_edge.reshape(1, D))


# speedup vs baseline: 16.4334x; 1.0559x over previous
import functools

import jax
import jax.numpy as jnp
from jax import lax
from jax.experimental import pallas as pl
from jax.experimental.pallas import tpu as pltpu
from jax.experimental.pallas import tpu_sc as plsc

N = 10000
E = 320000
D = 128
DE = 16
DW = D + 16

NC = 2
NS = 16
NW = NC * NS
EPW = E // NW
B = 80
NB = EPW // B
CH = 5
RPT = 632
RLAST = N - (NS - 1) * RPT

_mesh = plsc.VectorSubcoreMesh(core_axis_name="c", subcore_axis_name="s")


@functools.partial(
    pl.kernel,
    mesh=_mesh,
    out_type=jax.ShapeDtypeStruct((NW, N), jnp.float32),
    scratch_types=[
        pltpu.VMEM((EPW,), jnp.int32),
        pltpu.VMEM((N,), jnp.float32),
    ],
    compiler_params=pltpu.CompilerParams(needs_layout_passes=False),
)
def _deg_kernel(dst_hbm, out_hbm, dst_v, hist_v):
    cid = lax.axis_index("c")
    sid = lax.axis_index("s")
    wid = cid * NS + sid
    pltpu.sync_copy(dst_hbm.at[pl.ds(wid * EPW, EPW)], dst_v)

    zeros = jnp.zeros((16,), jnp.float32)

    def zbody(i, carry):
        hist_v[pl.ds(i * 16, 16)] = zeros
        return carry

    lax.fori_loop(0, N // 16, zbody, 0)

    ones = jnp.ones((16,), jnp.float32)

    def body(g, carry):
        idx = dst_v[pl.ds(g * 16, 16)]
        plsc.addupdate_scatter(hist_v, [idx], ones)
        return carry

    lax.fori_loop(0, EPW // 16, body, 0)
    pltpu.sync_copy(hist_v, out_hbm.at[wid])


_BLK = 1000
_GRID = N // _BLK


def _prep_body(degp, nfeat, wn, bn, rw, y, resid, norm):
    degs = jnp.sum(degp[...], axis=0) + 1.0
    nv = lax.rsqrt(degs)
    nv = nv * (1.5 - 0.5 * degs * nv * nv)
    inv = nv * nv
    x = nfeat[...]
    h = jnp.dot(x, wn[...], preferred_element_type=jnp.float32) + bn[...]
    y[:, :D] = nv[:, None] * x
    y[:, D:] = jnp.concatenate(
        [nv[:, None], jnp.zeros((N, DW - D - 1), jnp.float32)], axis=1)
    resid[...] = (h + rw[...]) * inv[:, None]
    norm[...] = nv[:, None]


def _prep(degp, nfeat, wn, bn, rw):
    return pl.pallas_call(
        _prep_body,
        out_shape=[
            jax.ShapeDtypeStruct((N, DW), jnp.float32),
            jax.ShapeDtypeStruct((N, D), jnp.float32),
            jax.ShapeDtypeStruct((N, 1), jnp.float32),
        ],
    )(degp, nfeat, wn, bn, rw)


@functools.partial(
    pl.kernel,
    mesh=_mesh,
    out_type=(
        jax.ShapeDtypeStruct((NC * N, DW), jnp.float32),
        jax.ShapeDtypeStruct((NC * N, DE), jnp.float32),
    ),
    scratch_types=[
        pltpu.VMEM((2, CH, B), jnp.int32),
        pltpu.VMEM((2, CH, B), jnp.int32),
        pltpu.VMEM((2, B, DW), jnp.float32),
        pltpu.VMEM((2, B * DE), jnp.float32),
        pltpu.VMEM((2, B, DE), jnp.float32),
        pltpu.VMEM_SHARED((N, DW), jnp.float32),
        pltpu.VMEM_SHARED((N, DE), jnp.float32),
        pltpu.SemaphoreType.DMA,
        pltpu.SemaphoreType.DMA,
        pltpu.SemaphoreType.DMA,
        pltpu.SemaphoreType.DMA,
    ],
    compiler_params=pltpu.CompilerParams(needs_layout_passes=False, use_tc_tiling_on_sc=False),
)
def _edge_kernel(src_hbm, dst_hbm, y_hbm, efeat_hbm, z144_hbm, z16_hbm,
                 g_out, ef_out, src_v, dst_v, ybuf, efbuf, build,
                 g_acc, ef_acc, sem_g, sem_e, sem_a, sem_i):
    cid = lax.axis_index("c")
    sid = lax.axis_index("s")
    wid = cid * NS + sid
    ebase = wid * EPW

    rbase = pl.multiple_of(sid * RPT, 8)

    @pl.when(sid < NS - 1)
    def _():
        pltpu.sync_copy(z144_hbm, g_acc.at[pl.ds(rbase, RPT)])
        pltpu.sync_copy(z16_hbm, ef_acc.at[pl.ds(rbase, RPT)])

    @pl.when(sid == NS - 1)
    def _():
        pltpu.sync_copy(z144_hbm.at[pl.ds(0, RLAST)],
                        g_acc.at[pl.ds((NS - 1) * RPT, RLAST)])
        pltpu.sync_copy(z16_hbm.at[pl.ds(0, RLAST)],
                        ef_acc.at[pl.ds((NS - 1) * RPT, RLAST)])

    plsc.subcore_barrier()

    pltpu.sync_copy(src_hbm.at[wid, pl.ds(0, CH)], src_v.at[0])
    pltpu.sync_copy(dst_hbm.at[wid, pl.ds(0, CH)], dst_v.at[0])
    pltpu.async_copy(y_hbm.at[src_v.at[0, 0]], ybuf.at[0], sem_g)
    pltpu.async_copy(efeat_hbm.at[pl.ds(ebase * DE, B * DE)], efbuf.at[0],
                     sem_e)

    iota16 = lax.iota(jnp.int32, 16)
    col_n = jnp.full((16,), D, jnp.int32)

    def body(b, carry):
        s = lax.rem(b, 2)
        c = lax.div(b, CH)
        cs = lax.rem(c, 2)
        bi = lax.rem(b, CH)

        pltpu.make_async_copy(y_hbm.at[src_v.at[cs, bi]], ybuf.at[s],
                              sem_g).wait()
        pltpu.make_async_copy(efeat_hbm.at[pl.ds(ebase * DE, B * DE)],
                              efbuf.at[s], sem_e).wait()

        @pl.when(b > 0)
        def _():
            pltpu.make_async_copy(ybuf.at[1 - s], g_acc.at[dst_v.at[cs, bi]],
                                  sem_a).wait()
            pltpu.make_async_copy(build.at[1 - s],
                                  ef_acc.at[dst_v.at[cs, bi]], sem_a).wait()

        @pl.when((bi == 0) & (b + CH < NB))
        def _():
            pltpu.async_copy(src_hbm.at[wid, pl.ds((c + 1) * CH, CH)],
                             src_v.at[1 - cs], sem_i)
            pltpu.async_copy(dst_hbm.at[wid, pl.ds((c + 1) * CH, CH)],
                             dst_v.at[1 - cs], sem_i)

        @pl.when((bi == CH - 1) & (b + 1 < NB))
        def _():
            pltpu.make_async_copy(src_hbm.at[wid, pl.ds(0, CH)],
                                  src_v.at[1 - cs], sem_i).wait()
            pltpu.make_async_copy(dst_hbm.at[wid, pl.ds(0, CH)],
                                  dst_v.at[1 - cs], sem_i).wait()

        @pl.when(b + 1 < NB)
        def _():
            c1 = lax.div(b + 1, CH)
            pltpu.async_copy(
                y_hbm.at[src_v.at[lax.rem(c1, 2), lax.rem(b + 1, CH)]],
                ybuf.at[1 - s], sem_g)
            pltpu.async_copy(
                efeat_hbm.at[pl.ds((ebase + (b + 1) * B) * DE, B * DE)],
                efbuf.at[1 - s], sem_e)

        for g in range(B // 16):
            rows = iota16 + g * 16
            sv = plsc.load_gather(ybuf.at[s], [rows, col_n])
            flat_base = iota16 * DE + g * 16 * DE
            for j in range(DE):
                jv = jnp.full((16,), j, jnp.int32)
                col = plsc.load_gather(efbuf.at[s], [flat_base + j])
                plsc.store_scatter(build.at[s], [rows, jv], col * sv)

        pltpu.async_copy(ybuf.at[s], g_acc.at[dst_v.at[cs, bi]], sem_a,
                         add=True)
        pltpu.async_copy(build.at[s], ef_acc.at[dst_v.at[cs, bi]], sem_a,
                         add=True)
        return carry

    lax.fori_loop(0, NB, body, 0)

    ls = lax.rem(NB - 1, 2)
    lcs = lax.rem(lax.div(NB - 1, CH), 2)
    lbi = lax.rem(NB - 1, CH)
    pltpu.make_async_copy(ybuf.at[ls], g_acc.at[dst_v.at[lcs, lbi]],
                          sem_a).wait()
    pltpu.make_async_copy(build.at[ls], ef_acc.at[dst_v.at[lcs, lbi]],
                          sem_a).wait()

    plsc.subcore_barrier()
    obase = pl.multiple_of(cid * N + sid * RPT, 8)

    @pl.when(sid < NS - 1)
    def _():
        pltpu.sync_copy(g_acc.at[pl.ds(rbase, RPT)],
                        g_out.at[pl.ds(obase, RPT)])
        pltpu.sync_copy(ef_acc.at[pl.ds(rbase, RPT)],
                        ef_out.at[pl.ds(obase, RPT)])

    @pl.when(sid == NS - 1)
    def _():
        lb = (NS - 1) * RPT
        pltpu.sync_copy(g_acc.at[pl.ds(lb, RLAST)],
                        g_out.at[pl.ds(cid * N + lb, RLAST)])
        pltpu.sync_copy(ef_acc.at[pl.ds(lb, RLAST)],
                        ef_out.at[pl.ds(cid * N + lb, RLAST)])


def _final_body(g0, g1, ef0, ef1, norm, resid, wn, we, bn, be, out):
    g = g0[...] + g1[...]
    ef = ef0[...] + ef1[...]
    t = g[:, D:D + 1]
    acc = jnp.dot(g[:, :D], wn[...], preferred_element_type=jnp.float32)
    acc += jnp.dot(ef, we[...], preferred_element_type=jnp.float32)
    acc += t * (bn[...] + be[...])
    out[...] = norm[...] * acc + resid[...]


def _final(gp, efp, norm, resid, wn, we, bn, be):
    return pl.pallas_call(
        _final_body,
        out_shape=jax.ShapeDtypeStruct((N, D), jnp.float32),
    )(gp[:N], gp[N:], efp[:N], efp[N:], norm, resid, wn, we, bn, be)


def kernel(nfeat, edge_index, efeat, W_node, b_node, W_edge, b_edge,
           res_weight):
    src = edge_index[0]
    dst = edge_index[1]

    degp = _deg_kernel(dst)
    y, resid, norm = _prep(degp, nfeat, W_node, b_node.reshape(1, D),
                           res_weight)
    z144 = jnp.zeros((RPT, DW), jnp.float32)
    z16 = jnp.zeros((RPT, DE), jnp.float32)
    gp, efp = _edge_kernel(src.reshape(NW, NB, B), dst.reshape(NW, NB, B),
                           y, efeat.reshape(E * DE), z144, z16)
    return _final(gp, efp, norm, resid, W_node, W_edge,
                  b_node.reshape(1, D), b_edge.reshape(1, D))


# speedup vs baseline: 17.2961x; 1.0525x over previous
import functools

import jax
import jax.numpy as jnp
from jax import lax
from jax.experimental import pallas as pl
from jax.experimental.pallas import tpu as pltpu
from jax.experimental.pallas import tpu_sc as plsc

N = 10000
E = 320000
D = 128
DE = 16
DW = D + 16

NC = 2
NS = 16
NW = NC * NS
EPW = E // NW
B = 80
NB = EPW // B
CH = 5
RPT = 632
RLAST = N - (NS - 1) * RPT

_mesh = plsc.VectorSubcoreMesh(core_axis_name="c", subcore_axis_name="s")


@functools.partial(
    pl.kernel,
    mesh=_mesh,
    out_type=jax.ShapeDtypeStruct((NW, N), jnp.float32),
    scratch_types=[
        pltpu.VMEM((EPW,), jnp.int32),
        pltpu.VMEM((N,), jnp.float32),
    ],
    compiler_params=pltpu.CompilerParams(needs_layout_passes=False),
)
def _deg_kernel(dst_hbm, out_hbm, dst_v, hist_v):
    cid = lax.axis_index("c")
    sid = lax.axis_index("s")
    wid = cid * NS + sid
    pltpu.sync_copy(dst_hbm.at[pl.ds(wid * EPW, EPW)], dst_v)

    zeros = jnp.zeros((16,), jnp.float32)

    def zbody(i, carry):
        hist_v[pl.ds(i * 16, 16)] = zeros
        return carry

    lax.fori_loop(0, N // 16, zbody, 0)

    ones = jnp.ones((16,), jnp.float32)

    def body(g, carry):
        idx = dst_v[pl.ds(g * 16, 16)]
        plsc.addupdate_scatter(hist_v, [idx], ones)
        return carry

    lax.fori_loop(0, EPW // 16, body, 0)
    pltpu.sync_copy(hist_v, out_hbm.at[wid])


_BLK = 1000
_GRID = N // _BLK


def _prep_body(degp, nfeat, wn, bn, rw, y, resid, norm):
    degs = jnp.sum(degp[...], axis=0) + 1.0
    nv = lax.rsqrt(degs)
    nv = nv * (1.5 - 0.5 * degs * nv * nv)
    inv = nv * nv
    x = nfeat[...]
    h = jnp.dot(x, wn[...], preferred_element_type=jnp.float32) + bn[...]
    y[:, :D] = nv[:, None] * x
    y[:, D:] = jnp.concatenate(
        [nv[:, None], jnp.zeros((N, DW - D - 1), jnp.float32)], axis=1)
    resid[...] = (h + rw[...]) * inv[:, None]
    norm[...] = nv[:, None]


def _prep(degp, nfeat, wn, bn, rw):
    return pl.pallas_call(
        _prep_body,
        out_shape=[
            jax.ShapeDtypeStruct((N, DW), jnp.float32),
            jax.ShapeDtypeStruct((N, D), jnp.float32),
            jax.ShapeDtypeStruct((N, 1), jnp.float32),
        ],
    )(degp, nfeat, wn, bn, rw)


@functools.partial(
    pl.kernel,
    mesh=_mesh,
    out_type=(
        jax.ShapeDtypeStruct((NC * N, DW), jnp.float32),
        jax.ShapeDtypeStruct((NC * N, DE), jnp.float32),
    ),
    scratch_types=[
        pltpu.VMEM((2, CH * B), jnp.int32),
        pltpu.VMEM((2, CH * B), jnp.int32),
        pltpu.VMEM((2, CH, B), jnp.int32),
        pltpu.VMEM((2, B, DW), jnp.float32),
        pltpu.VMEM((2, B, DE), jnp.float32),
        pltpu.VMEM((2, B, DE), jnp.float32),
        pltpu.VMEM_SHARED((N, DW), jnp.float32),
        pltpu.VMEM_SHARED((N, DE), jnp.float32),
        pltpu.SemaphoreType.DMA,
        pltpu.SemaphoreType.DMA,
        pltpu.SemaphoreType.DMA,
        pltpu.SemaphoreType.DMA,
    ],
    compiler_params=pltpu.CompilerParams(needs_layout_passes=False, use_tc_tiling_on_sc=False),
)
def _edge_kernel(src_hbm, dst_hbm, y_hbm, efeat_hbm, z144_hbm, z16_hbm,
                 g_out, ef_out, src_v, dst_st, dst_v, ybuf, efbuf, build,
                 g_acc, ef_acc, sem_g, sem_e, sem_a, sem_i):
    cid = lax.axis_index("c")
    sid = lax.axis_index("s")
    wid = cid * NS + sid
    ebase = wid * EPW

    rbase = pl.multiple_of(sid * RPT, 8)

    @pl.when(sid < NS - 1)
    def _():
        pltpu.sync_copy(z144_hbm, g_acc.at[pl.ds(rbase, RPT)])
        pltpu.sync_copy(z16_hbm, ef_acc.at[pl.ds(rbase, RPT)])

    @pl.when(sid == NS - 1)
    def _():
        pltpu.sync_copy(z144_hbm.at[pl.ds(0, RLAST)],
                        g_acc.at[pl.ds((NS - 1) * RPT, RLAST)])
        pltpu.sync_copy(z16_hbm.at[pl.ds(0, RLAST)],
                        ef_acc.at[pl.ds((NS - 1) * RPT, RLAST)])

    plsc.subcore_barrier()

    def stage_dst_rows(slot):
        for r in range(CH):
            for k in range(B // 16):
                dst_v[slot, r, pl.ds(k * 16, 16)] = (
                    dst_st[slot, pl.ds(r * B + k * 16, 16)])

    pltpu.sync_copy(src_hbm.at[pl.ds(ebase, CH * B)], src_v.at[0])
    pltpu.sync_copy(dst_hbm.at[pl.ds(ebase, CH * B)], dst_st.at[0])
    stage_dst_rows(0)
    pltpu.async_copy(y_hbm.at[src_v.at[0, pl.ds(0, B)]], ybuf.at[0], sem_g)
    pltpu.async_copy(efeat_hbm.at[pl.ds(ebase, B)], efbuf.at[0], sem_e)

    iota16 = lax.iota(jnp.int32, 16)
    col_n = jnp.full((16,), D, jnp.int32)

    def body(b, carry):
        s = lax.rem(b, 2)
        c = lax.div(b, CH)
        cs = lax.rem(c, 2)
        bi = lax.rem(b, CH)

        pltpu.make_async_copy(y_hbm.at[src_v.at[cs, pl.ds(bi * B, B)]],
                              ybuf.at[s], sem_g).wait()
        pltpu.make_async_copy(efeat_hbm.at[pl.ds(ebase, B)],
                              efbuf.at[s], sem_e).wait()

        @pl.when(b > 0)
        def _():
            pltpu.make_async_copy(ybuf.at[1 - s], g_acc.at[dst_v.at[cs, bi]],
                                  sem_a).wait()
            pltpu.make_async_copy(build.at[1 - s],
                                  ef_acc.at[dst_v.at[cs, bi]], sem_a).wait()

        @pl.when((bi == 0) & (b + CH < NB))
        def _():
            nb = ebase + (c + 1) * CH * B
            pltpu.async_copy(src_hbm.at[pl.ds(nb, CH * B)],
                             src_v.at[1 - cs], sem_i)
            pltpu.async_copy(dst_hbm.at[pl.ds(nb, CH * B)],
                             dst_st.at[1 - cs], sem_i)

        @pl.when((bi == CH - 1) & (b + 1 < NB))
        def _():
            pltpu.make_async_copy(src_hbm.at[pl.ds(0, CH * B)],
                                  src_v.at[1 - cs], sem_i).wait()
            pltpu.make_async_copy(dst_hbm.at[pl.ds(0, CH * B)],
                                  dst_st.at[1 - cs], sem_i).wait()
            stage_dst_rows(1 - cs)

        @pl.when(b + 1 < NB)
        def _():
            c1 = lax.div(b + 1, CH)
            pltpu.async_copy(
                y_hbm.at[src_v.at[lax.rem(c1, 2),
                                  pl.ds(lax.rem(b + 1, CH) * B, B)]],
                ybuf.at[1 - s], sem_g)
            pltpu.async_copy(efeat_hbm.at[pl.ds(ebase + (b + 1) * B, B)],
                             efbuf.at[1 - s], sem_e)

        for g in range(B // 16):
            rows = iota16 + g * 16
            sv = plsc.load_gather(ybuf.at[s], [rows, col_n])
            for j in range(DE):
                jv = jnp.full((16,), j, jnp.int32)
                col = plsc.load_gather(efbuf.at[s], [rows, jv])
                plsc.store_scatter(build.at[s], [rows, jv], col * sv)

        pltpu.async_copy(ybuf.at[s], g_acc.at[dst_v.at[cs, bi]], sem_a,
                         add=True)
        pltpu.async_copy(build.at[s], ef_acc.at[dst_v.at[cs, bi]], sem_a,
                         add=True)
        return carry

    lax.fori_loop(0, NB, body, 0)

    ls = lax.rem(NB - 1, 2)
    lcs = lax.rem(lax.div(NB - 1, CH), 2)
    lbi = lax.rem(NB - 1, CH)
    pltpu.make_async_copy(ybuf.at[ls], g_acc.at[dst_v.at[lcs, lbi]],
                          sem_a).wait()
    pltpu.make_async_copy(build.at[ls], ef_acc.at[dst_v.at[lcs, lbi]],
                          sem_a).wait()

    plsc.subcore_barrier()
    obase = pl.multiple_of(cid * N + sid * RPT, 8)

    @pl.when(sid < NS - 1)
    def _():
        pltpu.sync_copy(g_acc.at[pl.ds(rbase, RPT)],
                        g_out.at[pl.ds(obase, RPT)])
        pltpu.sync_copy(ef_acc.at[pl.ds(rbase, RPT)],
                        ef_out.at[pl.ds(obase, RPT)])

    @pl.when(sid == NS - 1)
    def _():
        lb = (NS - 1) * RPT
        pltpu.sync_copy(g_acc.at[pl.ds(lb, RLAST)],
                        g_out.at[pl.ds(cid * N + lb, RLAST)])
        pltpu.sync_copy(ef_acc.at[pl.ds(lb, RLAST)],
                        ef_out.at[pl.ds(cid * N + lb, RLAST)])


def _final_body(g0, g1, ef0, ef1, norm, resid, wn, we, bn, be, out):
    g = g0[...] + g1[...]
    ef = ef0[...] + ef1[...]
    t = g[:, D:D + 1]
    acc = jnp.dot(g[:, :D], wn[...], preferred_element_type=jnp.float32)
    acc += jnp.dot(ef, we[...], preferred_element_type=jnp.float32)
    acc += t * (bn[...] + be[...])
    out[...] = norm[...] * acc + resid[...]


def _final(gp, efp, norm, resid, wn, we, bn, be):
    return pl.pallas_call(
        _final_body,
        grid=(1,),
        in_specs=[
            pl.BlockSpec((N, DW), lambda i: (0, 0)),
            pl.BlockSpec((N, DW), lambda i: (1, 0)),
            pl.BlockSpec((N, DE), lambda i: (0, 0)),
            pl.BlockSpec((N, DE), lambda i: (1, 0)),
            pl.BlockSpec((N, 1), lambda i: (0, 0)),
            pl.BlockSpec((N, D), lambda i: (0, 0)),
            pl.BlockSpec((D, D), lambda i: (0, 0)),
            pl.BlockSpec((DE, D), lambda i: (0, 0)),
            pl.BlockSpec((D,), lambda i: (0,)),
            pl.BlockSpec((D,), lambda i: (0,)),
        ],
        out_specs=pl.BlockSpec((N, D), lambda i: (0, 0)),
        out_shape=jax.ShapeDtypeStruct((N, D), jnp.float32),
    )(gp, gp, efp, efp, norm, resid, wn, we, bn, be)


def kernel(nfeat, edge_index, efeat, W_node, b_node, W_edge, b_edge,
           res_weight):
    src = edge_index[0]
    dst = edge_index[1]
    degp = _deg_kernel(dst)
    y, resid, norm = _prep(degp, nfeat, W_node, b_node, res_weight)
    z144 = jnp.zeros((RPT, DW), jnp.float32)
    z16 = jnp.zeros((RPT, DE), jnp.float32)
    gp, efp = _edge_kernel(src, dst, y, efeat, z144, z16)
    return _final(gp, efp, norm, resid, W_node, W_edge, b_node, b_edge)
